# baseline (speedup 1.0000x reference)
import functools

import jax
import jax.numpy as jnp
from jax import lax
from jax.experimental import pallas as pl
from jax.experimental.pallas import tpu as pltpu
from jax.experimental.pallas import tpu_sc as plsc

N_NODES = 10000
N_PAD = 10240
D = 128
NC = 2
NS = 16
NW = NC * NS
CHUNK = 128
N_EDGES = 320000
EDGES_PER_TILE = N_EDGES // NW
EDGES_PAD_TILE = 10240
NCHUNKS = EDGES_PAD_TILE // CHUNK
BLKC = 10
NBLK = NCHUNKS // BLKC
SRC_STAGE = EDGES_PAD_TILE + 2 * CHUNK
ROWS_PER_TILE = N_PAD // NS


def _sc_aggregate(E_src, E_dst, X):
    mesh = plsc.VectorSubcoreMesh(
        core_axis_name="c", subcore_axis_name="s", num_cores=NC, num_subcores=NS
    )

    @functools.partial(
        pl.kernel,
        mesh=mesh,
        out_type=jax.ShapeDtypeStruct((NC, N_PAD, D), jnp.float32),
        scratch_types=[
            pltpu.VMEM((SRC_STAGE,), jnp.int32),
            pltpu.VMEM((BLKC, CHUNK), jnp.int32),
            pltpu.VMEM((2, CHUNK, D), jnp.float32),
            pltpu.VMEM_SHARED((N_PAD, D), jnp.float32),
            pltpu.SemaphoreType.DMA,
            pltpu.SemaphoreType.DMA,
        ],
    )
    def k(es_hbm, ed_hbm, x_hbm, out_hbm, src_v, dst_v, rows_v,
          agg_s, gsem_a, gsem_b):
        rows_a = rows_v.at[0]
        rows_b = rows_v.at[1]
        c = lax.axis_index("c")
        s = lax.axis_index("s")
        wid = c * NS + s
        pltpu.sync_copy(es_hbm.at[wid], src_v)
        zero16 = jnp.zeros((16,), jnp.float32)

        def zstore(r, carry):
            for t in range(D // 16):
                rows_v[0, r, pl.ds(t * 16, 16)] = zero16
            return carry

        lax.fori_loop(0, CHUNK, zstore, 0)
        r0 = s * ROWS_PER_TILE

        def zcopy(t, carry):
            pltpu.sync_copy(rows_a, agg_s.at[pl.ds(r0 + t * CHUNK, CHUNK)])
            return carry

        lax.fori_loop(0, ROWS_PER_TILE // CHUNK, zcopy, 0)
        plsc.subcore_barrier()

        def g_issue(j, buf, sem):
            pltpu.async_copy(
                x_hbm.at[src_v.at[pl.ds(j * CHUNK, CHUNK)]], buf, sem)

        def g_drain(j, buf, sem):
            pltpu.make_async_copy(
                x_hbm.at[src_v.at[pl.ds(j * CHUNK, CHUNK)]], buf, sem).wait()

        g_issue(0, rows_a, gsem_a)
        g_issue(1, rows_b, gsem_b)

        def block(k_, carry):
            pltpu.sync_copy(ed_hbm.at[wid, k_], dst_v)
            for i2 in range(BLKC // 2):
                j = k_ * BLKC + 2 * i2
                g_drain(j, rows_a, gsem_a)
                pltpu.sync_copy(rows_a, agg_s.at[dst_v.at[2 * i2]], add=True)
                g_issue(j + 2, rows_a, gsem_a)
                g_drain(j + 1, rows_b, gsem_b)
                pltpu.sync_copy(
                    rows_b, agg_s.at[dst_v.at[2 * i2 + 1]], add=True)
                g_issue(j + 3, rows_b, gsem_b)
            return carry

        lax.fori_loop(0, NBLK, block, 0)
        g_drain(NCHUNKS, rows_a, gsem_a)
        g_drain(NCHUNKS + 1, rows_b, gsem_b)
        plsc.subcore_barrier()
        pltpu.sync_copy(
            agg_s.at[pl.ds(r0, ROWS_PER_TILE)],
            out_hbm.at[c, pl.ds(r0, ROWS_PER_TILE)],
        )

    return k(E_src, E_dst, X)


def _tc_finish(P, W, b2):
    BLK = 1000
    grid = (N_NODES // BLK,)

    def body(p_ref, w_ref, b_ref, o_ref):
        a = p_ref[0] + p_ref[1]
        acc = jnp.dot(a, w_ref[...], preferred_element_type=jnp.float32)
        o_ref[...] = jnp.maximum(acc + b_ref[...], 0.0)

    return pl.pallas_call(
        body,
        grid=grid,
        in_specs=[
            pl.BlockSpec((2, BLK, D), lambda i: (0, i, 0)),
            pl.BlockSpec((D, D), lambda i: (0, 0)),
            pl.BlockSpec((1, D), lambda i: (0, 0)),
        ],
        out_specs=pl.BlockSpec((BLK, D), lambda i: (i, 0)),
        out_shape=jax.ShapeDtypeStruct((N_NODES, D), jnp.float32),
    )(P, W, b2)


def kernel(V, E, X, W, b):
    E0 = E[0].reshape(NW, EDGES_PER_TILE)
    E1 = E[1].reshape(NW, EDGES_PER_TILE)
    src_pad = jnp.zeros((NW, SRC_STAGE - EDGES_PER_TILE), jnp.int32)
    dst_pad = jnp.broadcast_to(
        jnp.arange(N_NODES, N_PAD, dtype=jnp.int32),
        (NW, EDGES_PAD_TILE - EDGES_PER_TILE))
    E_src = jnp.concatenate([E0, src_pad], axis=1)
    E_dst = jnp.concatenate([E1, dst_pad], axis=1).reshape(
        NW, NBLK, BLKC, CHUNK)
    P = _sc_aggregate(E_src, E_dst, X)
    return _tc_finish(P, W, b.reshape(1, D))


# speedup vs baseline: 4.6779x; 4.6779x over previous
import functools

import jax
import jax.numpy as jnp
from jax import lax
from jax.experimental import pallas as pl
from jax.experimental.pallas import tpu as pltpu
from jax.experimental.pallas import tpu_sc as plsc

N_NODES = 10000
N_PAD = 10240
D = 128
NC = 2
NS = 16
NW = NC * NS
CHUNK = 80
N_EDGES = 320000
EDGES_PER_TILE = N_EDGES // NW
NCHUNKS = EDGES_PER_TILE // CHUNK
ROWS_PER_TILE = N_PAD // NS


def _sc_aggregate(E_src, E_dst, X):
    mesh = plsc.VectorSubcoreMesh(
        core_axis_name="c", subcore_axis_name="s", num_cores=NC, num_subcores=NS
    )

    @functools.partial(
        pl.kernel,
        mesh=mesh,
        out_type=jax.ShapeDtypeStruct((NC, N_PAD, D), jnp.float32),
        scratch_types=[
            pltpu.VMEM((EDGES_PER_TILE,), jnp.int32),
            pltpu.VMEM((NCHUNKS, CHUNK), jnp.int32),
            pltpu.VMEM((2, CHUNK, D), jnp.float32),
            pltpu.VMEM_SHARED((N_PAD, D), jnp.float32),
            pltpu.SemaphoreType.DMA,
            pltpu.SemaphoreType.DMA,
        ],
    )
    def k(es_hbm, ed_hbm, x_hbm, out_hbm, src_v, dst_v, rows_v,
          agg_s, gsem_a, gsem_b):
        rows_a = rows_v.at[0]
        rows_b = rows_v.at[1]
        c = lax.axis_index("c")
        s = lax.axis_index("s")
        wid = c * NS + s
        pltpu.sync_copy(es_hbm.at[wid], src_v)
        pltpu.sync_copy(ed_hbm.at[wid], dst_v)
        zero16 = jnp.zeros((16,), jnp.float32)

        def zstore(r, carry):
            for t in range(D // 16):
                rows_v[0, r, pl.ds(t * 16, 16)] = zero16
            return carry

        lax.fori_loop(0, CHUNK, zstore, 0)
        r0 = s * ROWS_PER_TILE

        def zcopy(t, carry):
            pltpu.sync_copy(rows_a, agg_s.at[pl.ds(r0 + t * CHUNK, CHUNK)])
            return carry

        lax.fori_loop(0, ROWS_PER_TILE // CHUNK, zcopy, 0)
        plsc.subcore_barrier()

        HALF = CHUNK // 2

        def g_issue(j, buf, sem):
            base = j * CHUNK
            pltpu.async_copy(
                x_hbm.at[src_v.at[pl.ds(base, HALF)]],
                buf.at[pl.ds(0, HALF)], sem)
            pltpu.async_copy(
                x_hbm.at[src_v.at[pl.ds(base + HALF, HALF)]],
                buf.at[pl.ds(HALF, HALF)], sem)

        def g_drain(j, buf, sem):
            pltpu.make_async_copy(
                x_hbm.at[src_v.at[pl.ds(j * CHUNK, CHUNK)]], buf, sem).wait()

        g_issue(0, rows_a, gsem_a)
        g_issue(1, rows_b, gsem_b)

        def body(i, carry):
            j = 2 * i
            g_drain(j, rows_a, gsem_a)
            pltpu.sync_copy(rows_a, agg_s.at[dst_v.at[j]], add=True)
            g_issue(j + 2, rows_a, gsem_a)
            g_drain(j + 1, rows_b, gsem_b)
            pltpu.sync_copy(rows_b, agg_s.at[dst_v.at[j + 1]], add=True)
            g_issue(j + 3, rows_b, gsem_b)
            return carry

        lax.fori_loop(0, (NCHUNKS - 3) // 2, body, 0)
        g_drain(NCHUNKS - 3, rows_a, gsem_a)
        pltpu.sync_copy(rows_a, agg_s.at[dst_v.at[NCHUNKS - 3]], add=True)
        g_issue(NCHUNKS - 1, rows_a, gsem_a)
        g_drain(NCHUNKS - 2, rows_b, gsem_b)
        pltpu.sync_copy(rows_b, agg_s.at[dst_v.at[NCHUNKS - 2]], add=True)
        g_drain(NCHUNKS - 1, rows_a, gsem_a)
        pltpu.sync_copy(rows_a, agg_s.at[dst_v.at[NCHUNKS - 1]], add=True)
        plsc.subcore_barrier()
        pltpu.sync_copy(
            agg_s.at[pl.ds(r0, ROWS_PER_TILE)],
            out_hbm.at[c, pl.ds(r0, ROWS_PER_TILE)],
        )

    return k(E_src, E_dst, X)


def _tc_finish(P, W, b2):
    BLK = 1000
    grid = (N_NODES // BLK,)

    def body(p_ref, w_ref, b_ref, o_ref):
        a = p_ref[0] + p_ref[1]
        acc = jnp.dot(a, w_ref[...], preferred_element_type=jnp.float32)
        o_ref[...] = jnp.maximum(acc + b_ref[...], 0.0)

    return pl.pallas_call(
        body,
        grid=grid,
        in_specs=[
            pl.BlockSpec((2, BLK, D), lambda i: (0, i, 0)),
            pl.BlockSpec((D, D), lambda i: (0, 0)),
            pl.BlockSpec((1, D), lambda i: (0, 0)),
        ],
        out_specs=pl.BlockSpec((BLK, D), lambda i: (i, 0)),
        out_shape=jax.ShapeDtypeStruct((N_NODES, D), jnp.float32),
    )(P, W, b2)


def kernel(V, E, X, W, b):
    E_src = E[0].reshape(NW, EDGES_PER_TILE)
    E_dst = E[1].reshape(NW, NCHUNKS, CHUNK)
    P = _sc_aggregate(E_src, E_dst, X)
    return _tc_finish(P, W, b.reshape(1, D))
